# baseline (speedup 1.0000x reference)
import functools

import jax
import jax.numpy as jnp
from jax import lax
from jax.experimental import pallas as pl
from jax.experimental.pallas import tpu as pltpu
from jax.experimental.pallas import tpu_sc as plsc

B, S, H = 1, 2048, 1024
E, K, F = 8, 2, 1408
T = B * S
P = T * K
BT = 256
NPAD = 6144
NBLK = NPAD // BT
NW = 32
CPW = P // NW
TPW = T // NW
LANES = 16
WPAD = 128


def _router_body(x_ref, gw_ref, logits_ref, idx_ref, w_ref, base_ref,
                 bexp_ref, nblk_ref):
    x = x_ref[...]
    gw = gw_ref[...]
    logits = lax.dot_general(x, gw, (((1,), (1,)), ((), ())),
                             preferred_element_type=jnp.float32)
    logits_ref[...] = logits
    m = jnp.max(logits, axis=1, keepdims=True)
    p = jnp.exp(logits - m)
    p = p / jnp.sum(p, axis=1, keepdims=True)
    a1 = jnp.argmax(p, axis=1).astype(jnp.int32)
    m1 = jnp.max(p, axis=1)
    col = lax.broadcasted_iota(jnp.int32, p.shape, 1)
    p2 = jnp.where(col == a1[:, None], -1.0, p)
    a2 = jnp.argmax(p2, axis=1).astype(jnp.int32)
    m2 = jnp.max(p2, axis=1)
    s = m1 + m2
    col2 = lax.broadcasted_iota(jnp.int32, (T, K), 1)
    idx_ref[...] = jnp.where(col2 == 0,
                             jnp.broadcast_to(a1[:, None], (T, K)),
                             jnp.broadcast_to(a2[:, None], (T, K)))
    w_ref[...] = jnp.where(col2 == 0,
                           jnp.broadcast_to((m1 / s)[:, None], (T, K)),
                           jnp.broadcast_to((m2 / s)[:, None], (T, K)))

    ecol = lax.broadcasted_iota(jnp.int32, (T, LANES), 1)
    ohs = ((ecol == a1[:, None]).astype(jnp.float32)
           + (ecol == a2[:, None]).astype(jnp.float32))
    trow = lax.broadcasted_iota(jnp.int32, (NW, T), 1) // TPW
    wrow = lax.broadcasted_iota(jnp.int32, (NW, T), 0)
    sel = (trow == wrow).astype(jnp.float32)
    chunk = lax.dot_general(sel, ohs, (((1,), (0,)), ((), ())),
                            preferred_element_type=jnp.float32)
    ltr = lax.broadcasted_iota(jnp.int32, (NW, NW), 0)
    ltc = lax.broadcasted_iota(jnp.int32, (NW, NW), 1)
    lt = (ltc < ltr).astype(jnp.float32)
    pre = lax.dot_general(lt, chunk, (((1,), (0,)), ((), ())),
                          preferred_element_type=jnp.float32)
    tot = jnp.sum(chunk, axis=0, keepdims=True)
    toti = tot.astype(jnp.int32)
    tot_pad = (((toti + (BT - 1)) >> 8) << 8).astype(jnp.float32)
    utr = lax.broadcasted_iota(jnp.int32, (LANES, LANES), 0)
    utc = lax.broadcasted_iota(jnp.int32, (LANES, LANES), 1)
    ut = (utr <= utc).astype(jnp.float32)
    pinc = lax.dot_general(tot_pad, ut, (((1,), (0,)), ((), ())),
                           preferred_element_type=jnp.float32)
    pstart = pinc - tot_pad
    base_ref[...] = (pre + pstart).astype(jnp.int32)

    pinci = pinc.astype(jnp.int32)
    brow = lax.broadcasted_iota(jnp.int32, (NW, LANES), 0)
    bcol = lax.broadcasted_iota(jnp.int32, (NW, LANES), 1)
    ge = ((brow * BT >= pinci) & (bcol < E)).astype(jnp.int32)
    be_raw = jnp.minimum(jnp.sum(ge, axis=1, keepdims=True), E - 1)
    last = jnp.sum(jnp.where(bcol[:1] == E - 1, pinci, 0), axis=1,
                   keepdims=True)
    used = last >> 8
    nblk_ref[...] = jnp.broadcast_to(used, (1, 8))

    present = (tot_pad.astype(jnp.int32) > 0).astype(jnp.int32)
    be_last = jnp.max(bcol[:1] * present * (bcol[:1] < E).astype(jnp.int32),
                      axis=1, keepdims=True)
    brow1 = lax.broadcasted_iota(jnp.int32, (NW, 1), 0)
    bei = jnp.where(brow1 < used, be_raw, be_last)
    sh = (ltc == ltr - 1).astype(jnp.float32)
    prev = lax.dot_general(sh, bei.astype(jnp.float32),
                           (((1,), (0,)), ((), ())),
                           preferred_element_type=jnp.float32).astype(jnp.int32)
    ch = ((brow1 == 0) | (bei != prev)).astype(jnp.int32)
    lti = (ltc <= ltr).astype(jnp.float32)
    order = lax.dot_general(lti, ch.astype(jnp.float32),
                            (((1,), (0,)), ((), ())),
                            preferred_element_type=jnp.float32
                            ).astype(jnp.int32) - 1
    slot = order - (order // 3) * 3
    nxcand = jnp.where((present > 0) & (bcol > bei) & (bcol < E), bcol, 99)
    nx = jnp.min(nxcand, axis=1, keepdims=True)
    has = (nx < 99).astype(jnp.int32)
    nx2cand = jnp.where((present > 0) & (bcol > nx) & (bcol < E), bcol, 99)
    nx2 = jnp.min(nx2cand, axis=1, keepdims=True)
    has2 = has * (nx2 < 99).astype(jnp.int32)
    nx = jnp.where(has > 0, nx, 0)
    nx2 = jnp.where(has2 > 0, nx2, 0)
    mcol = lax.broadcasted_iota(jnp.int32, (NW, 8), 1)
    bexp_ref[...] = (jnp.where(mcol == 0, bei, 0)
                     + jnp.where(mcol == 1, ch, 0)
                     + jnp.where(mcol == 2, nx, 0)
                     + jnp.where(mcol == 3, has, 0)
                     + jnp.where(mcol == 4, slot, 0)
                     + jnp.where(mcol == 5, nx2, 0)
                     + jnp.where(mcol == 6, has2, 0))


_router = pl.pallas_call(
    _router_body,
    out_shape=(
        jax.ShapeDtypeStruct((T, E), jnp.float32),
        jax.ShapeDtypeStruct((T, K), jnp.int32),
        jax.ShapeDtypeStruct((T, K), jnp.float32),
        jax.ShapeDtypeStruct((NW, LANES), jnp.int32),
        jax.ShapeDtypeStruct((NW, 8), jnp.int32),
        jax.ShapeDtypeStruct((1, 8), jnp.int32),
    ),
)


_sc_mesh = plsc.VectorSubcoreMesh(core_axis_name="c", subcore_axis_name="s")


@functools.partial(
    pl.kernel,
    out_type=(
        jax.ShapeDtypeStruct((P,), jnp.int32),
        jax.ShapeDtypeStruct((NPAD, H), jnp.float32),
        jax.ShapeDtypeStruct((NPAD, WPAD), jnp.float32),
    ),
    mesh=_sc_mesh,
    compiler_params=pltpu.CompilerParams(needs_layout_passes=False),
    scratch_types=[
        pltpu.VMEM((CPW,), jnp.int32),
        pltpu.VMEM((CPW,), jnp.int32),
        pltpu.VMEM((CPW,), jnp.float32),
        pltpu.VMEM((LANES,), jnp.int32),
        pltpu.VMEM((TPW,), jnp.int32),
        pltpu.VMEM((TPW,), jnp.int32),
        pltpu.VMEM((TPW, H), jnp.float32),
        pltpu.VMEM((TPW, WPAD), jnp.float32),
        pltpu.VMEM((TPW, WPAD), jnp.float32),
        pltpu.SemaphoreType.DMA,
        pltpu.SemaphoreType.DMA,
    ],
)
def _dispatch(e_hbm, w_hbm, x_hbm, base_hbm, pos_hbm, xs_hbm, ws_hbm,
              e_v, pos_v, w_v, base_v, idxe_v, idxo_v, rows_v, wre_v, wro_v,
              sem, sem2):
    wid = lax.axis_index("s") * 2 + lax.axis_index("c")
    lane = lax.iota(jnp.int32, 16)
    rows_cp = pltpu.async_copy(
        x_hbm.at[pl.ds(pl.multiple_of(wid * TPW, 8), TPW)], rows_v, sem)
    pltpu.sync_copy(e_hbm.at[pl.ds(pl.multiple_of(wid * CPW, 8), CPW)], e_v)
    pltpu.sync_copy(w_hbm.at[pl.ds(pl.multiple_of(wid * CPW, 8), CPW)], w_v)
    pltpu.sync_copy(base_hbm.at[wid], base_v)

    run = base_v[...]
    for u in range(CPW // LANES):
        ev = e_v[pl.ds(u * LANES, LANES)]
        posv = jnp.zeros((LANES,), jnp.int32)
        cnts = jnp.zeros((LANES,), jnp.int32)
        for e in range(E):
            oh = (ev == e).astype(jnp.int32)
            cs = plsc.cumsum(oh)
            base = jnp.sum(jnp.where(lane == e, run, jnp.zeros_like(run)))
            posv = posv + oh * (base + cs - 1)
            c = plsc.all_reduce_population_count(ev == e)
            cnts = jnp.where(lane == e, cnts + c, cnts)
        run = run + cnts
        pos_v[pl.ds(u * LANES, LANES)] = posv
    pos_cp = pltpu.async_copy(
        pos_v, pos_hbm.at[pl.ds(pl.multiple_of(wid * CPW, 8), CPW)], sem2)

    zcol = jnp.zeros((LANES,), jnp.int32)
    for u in range(TPW // LANES):
        bidx = u * 2 * LANES + 2 * lane
        idxe_v[pl.ds(u * LANES, LANES)] = plsc.load_gather(pos_v, [bidx])
        idxo_v[pl.ds(u * LANES, LANES)] = plsc.load_gather(pos_v, [bidx + 1])
        plsc.store_scatter(wre_v, [u * LANES + lane, zcol],
                           plsc.load_gather(w_v, [bidx]))
        plsc.store_scatter(wro_v, [u * LANES + lane, zcol],
                           plsc.load_gather(w_v, [bidx + 1]))
    rows_cp.wait()
    cps = [
        pltpu.async_copy(rows_v, xs_hbm.at[idxe_v], sem),
        pltpu.async_copy(rows_v, xs_hbm.at[idxo_v], sem),
        pltpu.async_copy(wre_v, ws_hbm.at[idxe_v], sem2),
        pltpu.async_copy(wro_v, ws_hbm.at[idxo_v], sem2),
    ]
    for cp in cps:
        cp.wait()
    pos_cp.wait()


def _mlp_body(meta_ref, nblk_ref, xs_ref, ws_ref, wg_hbm, wu_hbm, wd_hbm,
              out_ref, wgb, wub, wdb, sem0, sem1, sem2):
    b = pl.program_id(0)
    ch = meta_ref[b, 1]
    sl = meta_ref[b, 4]
    sems = (sem0, sem1, sem2)

    def issue(e, s):
        pltpu.async_copy(wg_hbm.at[e], wgb.at[s], sems[s])
        pltpu.async_copy(wu_hbm.at[e], wub.at[s], sems[s])
        pltpu.async_copy(wd_hbm.at[e], wdb.at[s], sems[s])

    def drain(s):
        pltpu.make_async_copy(wg_hbm.at[0], wgb.at[s], sems[s]).wait()
        pltpu.make_async_copy(wu_hbm.at[0], wub.at[s], sems[s]).wait()
        pltpu.make_async_copy(wd_hbm.at[0], wdb.at[s], sems[s]).wait()

    @pl.when(ch == 1)
    def _():
        @pl.when(b == 0)
        def _():
            issue(meta_ref[0, 0], 0)

            @pl.when(meta_ref[0, 3] == 1)
            def _():
                issue(meta_ref[0, 2], 1)

        for s in range(3):
            @pl.when(sl == s)
            def _(s=s):
                drain(s)

        @pl.when(meta_ref[b, 6] == 1)
        def _():
            nx2e = meta_ref[b, 5]
            for s in range(3):
                @pl.when(sl == s)
                def _(s=s):
                    issue(nx2e, (s + 2) % 3)

    @pl.when(b < nblk_ref[0])
    def _():
        xb = xs_ref[...]
        g = lax.dot_general(xb, wgb[sl], (((1,), (1,)), ((), ())),
                            preferred_element_type=jnp.float32)
        u = lax.dot_general(xb, wub[sl], (((1,), (1,)), ((), ())),
                            preferred_element_type=jnp.float32)
        h = g * jax.nn.sigmoid(g) * u
        d = lax.dot_general(h, wdb[sl], (((1,), (1,)), ((), ())),
                            preferred_element_type=jnp.float32)
        out_ref[...] = d * ws_ref[...][:, :1]


_mlp = pl.pallas_call(
    _mlp_body,
    grid_spec=pltpu.PrefetchScalarGridSpec(
        num_scalar_prefetch=2,
        grid=(NBLK,),
        in_specs=[
            pl.BlockSpec((BT, H), lambda b, be, nb: (b, 0)),
            pl.BlockSpec((BT, WPAD), lambda b, be, nb: (b, 0)),
            pl.BlockSpec(memory_space=pl.ANY),
            pl.BlockSpec(memory_space=pl.ANY),
            pl.BlockSpec(memory_space=pl.ANY),
        ],
        out_specs=pl.BlockSpec((BT, H), lambda b, be, nb: (b, 0)),
        scratch_shapes=[
            pltpu.VMEM((3, F, H), jnp.float32),
            pltpu.VMEM((3, F, H), jnp.float32),
            pltpu.VMEM((3, H, F), jnp.float32),
            pltpu.SemaphoreType.DMA,
            pltpu.SemaphoreType.DMA,
            pltpu.SemaphoreType.DMA,
        ],
    ),
    out_shape=jax.ShapeDtypeStruct((NPAD, H), jnp.float32),
    compiler_params=pltpu.CompilerParams(
        dimension_semantics=("arbitrary",),
        vmem_limit_bytes=100 * 1024 * 1024,
    ),
)


@functools.partial(
    pl.kernel,
    out_type=jax.ShapeDtypeStruct((T, H), jnp.float32),
    mesh=_sc_mesh,
    compiler_params=pltpu.CompilerParams(needs_layout_passes=False),
    scratch_types=[
        pltpu.VMEM((CPW,), jnp.int32),
        pltpu.VMEM((TPW,), jnp.int32),
        pltpu.VMEM((TPW // 2,), jnp.int32),
        pltpu.VMEM((TPW // 2,), jnp.int32),
        pltpu.VMEM((TPW, H), jnp.float32),
        pltpu.VMEM((TPW // 2, H), jnp.float32),
        pltpu.SemaphoreType.DMA,
    ],
)
def _combine(ys_hbm, pos_hbm, y_hbm, pos_v, idxe_v, idxo0_v, idxo1_v,
             out_v, odd_v, sem):
    wid = lax.axis_index("s") * 2 + lax.axis_index("c")
    lane = lax.iota(jnp.int32, 16)
    pltpu.sync_copy(pos_hbm.at[pl.ds(pl.multiple_of(wid * CPW, 8), CPW)], pos_v)
    for u in range(TPW // LANES):
        base = u * 2 * LANES + 2 * lane
        idxe_v[pl.ds(u * LANES, LANES)] = plsc.load_gather(pos_v, [base])
        odd = plsc.load_gather(pos_v, [base + 1])
        if u < 2:
            idxo0_v[pl.ds(u * LANES, LANES)] = odd
        else:
            idxo1_v[pl.ds((u - 2) * LANES, LANES)] = odd
    pltpu.async_copy(ys_hbm.at[idxe_v], out_v, sem).wait()
    for half, idxo in ((0, idxo0_v), (1, idxo1_v)):
        pltpu.async_copy(ys_hbm.at[idxo], odd_v, sem).wait()

        def row_add(i, _):
            for c in range(H // LANES):
                sl = pl.ds(c * LANES, LANES)
                out_v[half * (TPW // 2) + i, sl] = (
                    out_v[half * (TPW // 2) + i, sl] + odd_v[i, sl])
            return 0

        lax.fori_loop(0, TPW // 2, row_add, 0)
    pltpu.sync_copy(out_v, y_hbm.at[pl.ds(wid * TPW, TPW)])


@jax.jit
def kernel(hidden_states, gate_w, gate_proj_w, up_proj_w, down_proj_w):
    x = hidden_states.reshape(T, H)
    logits, idx, w, baser, meta, nblk = _router(x, gate_w)
    pos, xs, ws = _dispatch(idx.reshape(P), w.reshape(P), x, baser)
    ys = _mlp(meta, nblk.reshape(8), xs, ws,
              gate_proj_w, up_proj_w, down_proj_w)
    y = _combine(ys, pos)
    return y.reshape(B, S, H), logits


# speedup vs baseline: 1.0276x; 1.0276x over previous
import functools

import jax
import jax.numpy as jnp
from jax import lax
from jax.experimental import pallas as pl
from jax.experimental.pallas import tpu as pltpu
from jax.experimental.pallas import tpu_sc as plsc

B, S, H = 1, 2048, 1024
E, K, F = 8, 2, 1408
T = B * S
P = T * K
BT = 256
NPAD = 6144
NBLK = NPAD // BT
NW = 32
CPW = P // NW
TPW = T // NW
LANES = 16
WPAD = 128


def _router_body(x_ref, gw_ref, logits_ref, idx_ref, w_ref, base_ref,
                 bexp_ref, nblk_ref):
    x = x_ref[...]
    gw = gw_ref[...]
    logits = lax.dot_general(x, gw, (((1,), (1,)), ((), ())),
                             preferred_element_type=jnp.float32)
    logits_ref[...] = logits
    m = jnp.max(logits, axis=1, keepdims=True)
    p = jnp.exp(logits - m)
    p = p / jnp.sum(p, axis=1, keepdims=True)
    a1 = jnp.argmax(p, axis=1).astype(jnp.int32)
    m1 = jnp.max(p, axis=1)
    col = lax.broadcasted_iota(jnp.int32, p.shape, 1)
    p2 = jnp.where(col == a1[:, None], -1.0, p)
    a2 = jnp.argmax(p2, axis=1).astype(jnp.int32)
    m2 = jnp.max(p2, axis=1)
    s = m1 + m2
    col2 = lax.broadcasted_iota(jnp.int32, (T, K), 1)
    idx_ref[...] = jnp.where(col2 == 0,
                             jnp.broadcast_to(a1[:, None], (T, K)),
                             jnp.broadcast_to(a2[:, None], (T, K)))
    w_ref[...] = jnp.where(col2 == 0,
                           jnp.broadcast_to((m1 / s)[:, None], (T, K)),
                           jnp.broadcast_to((m2 / s)[:, None], (T, K)))

    ecol = lax.broadcasted_iota(jnp.int32, (T, LANES), 1)
    ohs = ((ecol == a1[:, None]).astype(jnp.float32)
           + (ecol == a2[:, None]).astype(jnp.float32))
    trow = lax.broadcasted_iota(jnp.int32, (NW, T), 1) // TPW
    wrow = lax.broadcasted_iota(jnp.int32, (NW, T), 0)
    sel = (trow == wrow).astype(jnp.float32)
    chunk = lax.dot_general(sel, ohs, (((1,), (0,)), ((), ())),
                            preferred_element_type=jnp.float32)
    ltr = lax.broadcasted_iota(jnp.int32, (NW, NW), 0)
    ltc = lax.broadcasted_iota(jnp.int32, (NW, NW), 1)
    lt = (ltc < ltr).astype(jnp.float32)
    pre = lax.dot_general(lt, chunk, (((1,), (0,)), ((), ())),
                          preferred_element_type=jnp.float32)
    tot = jnp.sum(chunk, axis=0, keepdims=True)
    toti = tot.astype(jnp.int32)
    tot_pad = (((toti + (BT - 1)) >> 8) << 8).astype(jnp.float32)
    utr = lax.broadcasted_iota(jnp.int32, (LANES, LANES), 0)
    utc = lax.broadcasted_iota(jnp.int32, (LANES, LANES), 1)
    ut = (utr <= utc).astype(jnp.float32)
    pinc = lax.dot_general(tot_pad, ut, (((1,), (0,)), ((), ())),
                           preferred_element_type=jnp.float32)
    pstart = pinc - tot_pad
    base_ref[...] = (pre + pstart).astype(jnp.int32)

    pinci = pinc.astype(jnp.int32)
    brow = lax.broadcasted_iota(jnp.int32, (NW, LANES), 0)
    bcol = lax.broadcasted_iota(jnp.int32, (NW, LANES), 1)
    ge = ((brow * BT >= pinci) & (bcol < E)).astype(jnp.int32)
    be_raw = jnp.minimum(jnp.sum(ge, axis=1, keepdims=True), E - 1)
    last = jnp.sum(jnp.where(bcol[:1] == E - 1, pinci, 0), axis=1,
                   keepdims=True)
    used = last >> 8
    nblk_ref[...] = jnp.broadcast_to(used, (1, 8))

    present = (tot_pad.astype(jnp.int32) > 0).astype(jnp.int32)
    be_last = jnp.max(bcol[:1] * present * (bcol[:1] < E).astype(jnp.int32),
                      axis=1, keepdims=True)
    brow1 = lax.broadcasted_iota(jnp.int32, (NW, 1), 0)
    bei = jnp.where(brow1 < used, be_raw, be_last)
    sh = (ltc == ltr - 1).astype(jnp.float32)
    prev = lax.dot_general(sh, bei.astype(jnp.float32),
                           (((1,), (0,)), ((), ())),
                           preferred_element_type=jnp.float32).astype(jnp.int32)
    ch = ((brow1 == 0) | (bei != prev)).astype(jnp.int32)
    lti = (ltc <= ltr).astype(jnp.float32)
    order = lax.dot_general(lti, ch.astype(jnp.float32),
                            (((1,), (0,)), ((), ())),
                            preferred_element_type=jnp.float32
                            ).astype(jnp.int32) - 1
    slot = order & 1
    nxcand = jnp.where((present > 0) & (bcol > bei) & (bcol < E), bcol, 99)
    nx = jnp.min(nxcand, axis=1, keepdims=True)
    has = (nx < 99).astype(jnp.int32)
    nx = jnp.where(has > 0, nx, 0)
    mcol = lax.broadcasted_iota(jnp.int32, (NW, 8), 1)
    bexp_ref[...] = (jnp.where(mcol == 0, bei, 0)
                     + jnp.where(mcol == 1, ch, 0)
                     + jnp.where(mcol == 2, nx, 0)
                     + jnp.where(mcol == 3, has, 0)
                     + jnp.where(mcol == 4, slot, 0))


_router = pl.pallas_call(
    _router_body,
    out_shape=(
        jax.ShapeDtypeStruct((T, E), jnp.float32),
        jax.ShapeDtypeStruct((T, K), jnp.int32),
        jax.ShapeDtypeStruct((T, K), jnp.float32),
        jax.ShapeDtypeStruct((NW, LANES), jnp.int32),
        jax.ShapeDtypeStruct((NW, 8), jnp.int32),
        jax.ShapeDtypeStruct((1, 8), jnp.int32),
    ),
)


_sc_mesh = plsc.VectorSubcoreMesh(core_axis_name="c", subcore_axis_name="s")


@functools.partial(
    pl.kernel,
    out_type=(
        jax.ShapeDtypeStruct((P,), jnp.int32),
        jax.ShapeDtypeStruct((NPAD, H), jnp.float32),
        jax.ShapeDtypeStruct((NPAD, WPAD), jnp.float32),
    ),
    mesh=_sc_mesh,
    compiler_params=pltpu.CompilerParams(needs_layout_passes=False),
    scratch_types=[
        pltpu.VMEM((CPW,), jnp.int32),
        pltpu.VMEM((CPW,), jnp.int32),
        pltpu.VMEM((CPW,), jnp.float32),
        pltpu.VMEM((LANES,), jnp.int32),
        pltpu.VMEM((TPW,), jnp.int32),
        pltpu.VMEM((TPW,), jnp.int32),
        pltpu.VMEM((TPW, H), jnp.float32),
        pltpu.VMEM((TPW, WPAD), jnp.float32),
        pltpu.VMEM((TPW, WPAD), jnp.float32),
        pltpu.SemaphoreType.DMA,
        pltpu.SemaphoreType.DMA,
    ],
)
def _dispatch(e_hbm, w_hbm, x_hbm, base_hbm, pos_hbm, xs_hbm, ws_hbm,
              e_v, pos_v, w_v, base_v, idxe_v, idxo_v, rows_v, wre_v, wro_v,
              sem, sem2):
    wid = lax.axis_index("s") * 2 + lax.axis_index("c")
    lane = lax.iota(jnp.int32, 16)
    rows_cp = pltpu.async_copy(
        x_hbm.at[pl.ds(pl.multiple_of(wid * TPW, 8), TPW)], rows_v, sem)
    pltpu.sync_copy(e_hbm.at[pl.ds(pl.multiple_of(wid * CPW, 8), CPW)], e_v)
    pltpu.sync_copy(w_hbm.at[pl.ds(pl.multiple_of(wid * CPW, 8), CPW)], w_v)
    pltpu.sync_copy(base_hbm.at[wid], base_v)

    run = base_v[...]
    for u in range(CPW // LANES):
        ev = e_v[pl.ds(u * LANES, LANES)]
        posv = jnp.zeros((LANES,), jnp.int32)
        cnts = jnp.zeros((LANES,), jnp.int32)
        for e in range(E):
            oh = (ev == e).astype(jnp.int32)
            cs = plsc.cumsum(oh)
            base = jnp.sum(jnp.where(lane == e, run, jnp.zeros_like(run)))
            posv = posv + oh * (base + cs - 1)
            c = plsc.all_reduce_population_count(ev == e)
            cnts = jnp.where(lane == e, cnts + c, cnts)
        run = run + cnts
        pos_v[pl.ds(u * LANES, LANES)] = posv
    pos_cp = pltpu.async_copy(
        pos_v, pos_hbm.at[pl.ds(pl.multiple_of(wid * CPW, 8), CPW)], sem2)

    zcol = jnp.zeros((LANES,), jnp.int32)
    for u in range(TPW // LANES):
        bidx = u * 2 * LANES + 2 * lane
        idxe_v[pl.ds(u * LANES, LANES)] = plsc.load_gather(pos_v, [bidx])
        idxo_v[pl.ds(u * LANES, LANES)] = plsc.load_gather(pos_v, [bidx + 1])
        plsc.store_scatter(wre_v, [u * LANES + lane, zcol],
                           plsc.load_gather(w_v, [bidx]))
        plsc.store_scatter(wro_v, [u * LANES + lane, zcol],
                           plsc.load_gather(w_v, [bidx + 1]))
    rows_cp.wait()
    cps = [
        pltpu.async_copy(rows_v, xs_hbm.at[idxe_v], sem),
        pltpu.async_copy(rows_v, xs_hbm.at[idxo_v], sem),
        pltpu.async_copy(wre_v, ws_hbm.at[idxe_v], sem2),
        pltpu.async_copy(wro_v, ws_hbm.at[idxo_v], sem2),
    ]
    for cp in cps:
        cp.wait()
    pos_cp.wait()


def _mlp_body(meta_ref, nblk_ref, xs_ref, ws_ref, wg_hbm, wu_hbm, wd_hbm,
              out_ref, wgb, wub, wdb, sem0, sem1):
    b = pl.program_id(0)
    ch = meta_ref[b, 1]
    sl = meta_ref[b, 4]

    def issue(e, s, sem):
        pltpu.async_copy(wg_hbm.at[e], wgb.at[s], sem)
        pltpu.async_copy(wu_hbm.at[e], wub.at[s], sem)
        pltpu.async_copy(wd_hbm.at[e], wdb.at[s], sem)

    def drain(s, sem):
        pltpu.make_async_copy(wg_hbm.at[0], wgb.at[s], sem).wait()
        pltpu.make_async_copy(wu_hbm.at[0], wub.at[s], sem).wait()
        pltpu.make_async_copy(wd_hbm.at[0], wdb.at[s], sem).wait()

    @pl.when(ch == 1)
    def _():
        @pl.when(b == 0)
        def _():
            issue(meta_ref[0, 0], 0, sem0)

        @pl.when(sl == 0)
        def _():
            drain(0, sem0)

        @pl.when(sl == 1)
        def _():
            drain(1, sem1)

        @pl.when(meta_ref[b, 3] == 1)
        def _():
            nxe = meta_ref[b, 2]

            @pl.when(sl == 0)
            def _():
                issue(nxe, 1, sem1)

            @pl.when(sl == 1)
            def _():
                issue(nxe, 0, sem0)

    @pl.when(b < nblk_ref[0])
    def _():
        xb = xs_ref[...]
        g = lax.dot_general(xb, wgb[sl], (((1,), (1,)), ((), ())),
                            preferred_element_type=jnp.float32)
        u = lax.dot_general(xb, wub[sl], (((1,), (1,)), ((), ())),
                            preferred_element_type=jnp.float32)
        h = g * jax.nn.sigmoid(g) * u
        d = lax.dot_general(h, wdb[sl], (((1,), (1,)), ((), ())),
                            preferred_element_type=jnp.float32)
        out_ref[...] = d * ws_ref[...][:, :1]


_mlp = pl.pallas_call(
    _mlp_body,
    grid_spec=pltpu.PrefetchScalarGridSpec(
        num_scalar_prefetch=2,
        grid=(NBLK,),
        in_specs=[
            pl.BlockSpec((BT, H), lambda b, be, nb: (b, 0)),
            pl.BlockSpec((BT, WPAD), lambda b, be, nb: (b, 0)),
            pl.BlockSpec(memory_space=pl.ANY),
            pl.BlockSpec(memory_space=pl.ANY),
            pl.BlockSpec(memory_space=pl.ANY),
        ],
        out_specs=pl.BlockSpec((BT, H), lambda b, be, nb: (b, 0)),
        scratch_shapes=[
            pltpu.VMEM((2, F, H), jnp.float32),
            pltpu.VMEM((2, F, H), jnp.float32),
            pltpu.VMEM((2, H, F), jnp.float32),
            pltpu.SemaphoreType.DMA,
            pltpu.SemaphoreType.DMA,
        ],
    ),
    out_shape=jax.ShapeDtypeStruct((NPAD, H), jnp.float32),
    compiler_params=pltpu.CompilerParams(
        dimension_semantics=("arbitrary",),
        vmem_limit_bytes=100 * 1024 * 1024,
    ),
)


@functools.partial(
    pl.kernel,
    out_type=jax.ShapeDtypeStruct((T, H), jnp.float32),
    mesh=_sc_mesh,
    compiler_params=pltpu.CompilerParams(needs_layout_passes=False),
    scratch_types=[
        pltpu.VMEM((CPW,), jnp.int32),
        pltpu.VMEM((TPW,), jnp.int32),
        pltpu.VMEM((TPW // 2,), jnp.int32),
        pltpu.VMEM((TPW // 2,), jnp.int32),
        pltpu.VMEM((TPW, H), jnp.float32),
        pltpu.VMEM((TPW // 2, H), jnp.float32),
        pltpu.SemaphoreType.DMA,
    ],
)
def _combine(ys_hbm, pos_hbm, y_hbm, pos_v, idxe_v, idxo0_v, idxo1_v,
             out_v, odd_v, sem):
    wid = lax.axis_index("s") * 2 + lax.axis_index("c")
    lane = lax.iota(jnp.int32, 16)
    pltpu.sync_copy(pos_hbm.at[pl.ds(pl.multiple_of(wid * CPW, 8), CPW)], pos_v)
    for u in range(TPW // LANES):
        base = u * 2 * LANES + 2 * lane
        idxe_v[pl.ds(u * LANES, LANES)] = plsc.load_gather(pos_v, [base])
        odd = plsc.load_gather(pos_v, [base + 1])
        if u < 2:
            idxo0_v[pl.ds(u * LANES, LANES)] = odd
        else:
            idxo1_v[pl.ds((u - 2) * LANES, LANES)] = odd
    pltpu.async_copy(ys_hbm.at[idxe_v], out_v, sem).wait()
    for half, idxo in ((0, idxo0_v), (1, idxo1_v)):
        pltpu.async_copy(ys_hbm.at[idxo], odd_v, sem).wait()

        def row_add(i, _):
            for c in range(H // LANES):
                sl = pl.ds(c * LANES, LANES)
                out_v[half * (TPW // 2) + i, sl] = (
                    out_v[half * (TPW // 2) + i, sl] + odd_v[i, sl])
            return 0

        lax.fori_loop(0, TPW // 2, row_add, 0)
    pltpu.sync_copy(out_v, y_hbm.at[pl.ds(wid * TPW, TPW)])


@jax.jit
def kernel(hidden_states, gate_w, gate_proj_w, up_proj_w, down_proj_w):
    x = hidden_states.reshape(T, H)
    logits, idx, w, baser, meta, nblk = _router(x, gate_w)
    pos, xs, ws = _dispatch(idx.reshape(P), w.reshape(P), x, baser)
    ys = _mlp(meta, nblk.reshape(8), xs, ws,
              gate_proj_w, up_proj_w, down_proj_w)
    y = _combine(ys, pos)
    return y.reshape(B, S, H), logits
